# Initial kernel scaffold; baseline (speedup 1.0000x reference)
import jax
import jax.numpy as jnp
from jax.experimental import pallas as pl


def kernel(h, nodeset, nb_nodes, nb_weights, Q_w, Q_b, W_w, W_b):
    raise NotImplementedError("write your pallas kernel here")


# speedup vs baseline: 2.6932x; 2.6932x over previous
import functools

import jax
import jax.numpy as jnp
from jax import lax
from jax.experimental import pallas as pl
from jax.experimental.pallas import tpu as pltpu
from jax.experimental.pallas import tpu_sc as plsc

_N = 100000
_D = 128
_B = 10000
_T = 32

_NC = 2
_NS = 16
_NW = _NC * _NS
_BPAD = 10240
_BPW = _BPAD // _NW
_IDXCHUNK = 128


def _leaky(x):
    return jnp.where(x >= 0, x, 0.01 * x)


def _hq_body(h_ref, qwt_ref, qb_ref, o_ref):
    x = jnp.dot(h_ref[...], qwt_ref[...], preferred_element_type=jnp.float32)
    o_ref[...] = _leaky(x + qb_ref[...])


def _hq_precompute(h, q_wt, q_b2):
    blk = 2000
    return pl.pallas_call(
        _hq_body,
        grid=(_N // blk,),
        in_specs=[
            pl.BlockSpec((blk, _D), lambda i: (i, 0)),
            pl.BlockSpec((_D, _D), lambda i: (0, 0)),
            pl.BlockSpec((1, _D), lambda i: (0, 0)),
        ],
        out_specs=pl.BlockSpec((blk, _D), lambda i: (i, 0)),
        out_shape=jax.ShapeDtypeStruct((_N, _D), jnp.float32),
    )(h, q_wt, q_b2)


def _out_body(hn_ref, agg_ref, w_ref, w1_ref, w2_ref, b_ref, o_ref):
    wsum = jnp.sum(w_ref[...], axis=1, keepdims=True)
    agg = agg_ref[...] / jnp.where(wsum == 0.0, 1.0, wsum)
    x = jnp.dot(hn_ref[...], w1_ref[...], preferred_element_type=jnp.float32)
    x = x + jnp.dot(agg, w2_ref[...], preferred_element_type=jnp.float32)
    x = _leaky(x + b_ref[...])
    nrm = jnp.sqrt(jnp.sum(x * x, axis=1, keepdims=True))
    o_ref[...] = x / jnp.where(nrm == 0.0, 1.0, nrm)


def _out_layer(hn, agg, w_p, w1t, w2t, w_b2):
    blk = 2048
    return pl.pallas_call(
        _out_body,
        grid=(_BPAD // blk,),
        in_specs=[
            pl.BlockSpec((blk, _D), lambda i: (i, 0)),
            pl.BlockSpec((blk, _D), lambda i: (i, 0)),
            pl.BlockSpec((blk, _T), lambda i: (i, 0)),
            pl.BlockSpec((_D, _D), lambda i: (0, 0)),
            pl.BlockSpec((_D, _D), lambda i: (0, 0)),
            pl.BlockSpec((1, _D), lambda i: (0, 0)),
        ],
        out_specs=pl.BlockSpec((blk, _D), lambda i: (i, 0)),
        out_shape=jax.ShapeDtypeStruct((_BPAD, _D), jnp.float32),
    )(hn, agg, w_p, w1t, w2t, w_b2)


def _wid():
    return lax.axis_index("s") * _NC + lax.axis_index("c")


def _chunked_row_gather(table_hbm, idx_v, idx_lo, dst_v, nrows, sem):
    copies = []
    for lo in range(0, nrows, _IDXCHUNK):
        n = min(_IDXCHUNK, nrows - lo)
        copies.append(pltpu.async_copy(
            table_hbm.at[idx_v.at[pl.ds(idx_lo + lo, n)]],
            dst_v.at[pl.ds(lo, n)], sem))
    return copies


def _sc_agg_body(hq_hbm, h_hbm, node_hbm, nb_hbm, w_hbm,
                 hn_out, agg_out,
                 node_v, nb_v, w_v, hn_v, agg_v, nbr_v,
                 sem_h, sem_nb, sem_w, sem0, sem1):
    base = _wid() * _BPW
    half = _BPW // 2
    pltpu.sync_copy(node_hbm.at[pl.ds(base, _BPW)], node_v)
    cp_nb = pltpu.async_copy(nb_hbm.at[pl.ds(base, _BPW)], nb_v, sem_nb)
    cp_w = pltpu.async_copy(w_hbm.at[pl.ds(base * _T, _BPW * _T)], w_v, sem_w)
    cp_nb.wait()
    cp_w.wait()

    sems = (sem0, sem1)
    pltpu.async_copy(hq_hbm.at[nb_v.at[0]], nbr_v.at[0], sem0)
    pltpu.async_copy(hq_hbm.at[nb_v.at[1]], nbr_v.at[1], sem1)

    def compute_row(i, local_i, buf):
        wr0 = w_v[pl.ds(i * _T, 16)]
        wr1 = w_v[pl.ds(i * _T + 16, 16)]
        acc = [jnp.zeros((16,), jnp.float32) for _ in range(8)]
        dnums = lax.GatherDimensionNumbers(
            offset_dims=(), collapsed_slice_dims=(0,), start_index_map=(0,))
        for t in range(_T):
            src = wr0 if t < 16 else wr1
            wt = lax.gather(src, jnp.full((16, 1), t % 16, jnp.int32),
                            dnums, slice_sizes=(1,),
                            mode=lax.GatherScatterMode.PROMISE_IN_BOUNDS)
            for c in range(8):
                acc[c] = acc[c] + wt * nbr_v[buf, t, pl.ds(c * 16, 16)]
        for c in range(8):
            agg_v[pl.ds(local_i * _D + c * 16, 16)] = acc[c]

    for hh in range(2):
        hlo = hh * half
        cps_h = _chunked_row_gather(h_hbm, node_v, hlo, hn_v, half, sem_h)

        def body(k, carry, hlo=hlo):
            for s in range(2):
                local_i = 2 * k + s
                i = hlo + local_i
                pltpu.make_async_copy(hq_hbm.at[nb_v.at[i]], nbr_v.at[s],
                                      sems[s]).wait()
                compute_row(i, local_i, s)

                @pl.when(i + 2 < _BPW)
                def _():
                    pltpu.async_copy(hq_hbm.at[nb_v.at[i + 2]], nbr_v.at[s],
                                     sems[s])
            return carry

        lax.fori_loop(0, half // 2, body, 0)

        for cp in cps_h:
            cp.wait()
        pltpu.sync_copy(hn_v, hn_out.at[pl.ds(base + hlo, half)])
        pltpu.sync_copy(agg_v,
                        agg_out.at[pl.ds((base + hlo) * _D, half * _D)])


def _sc_aggregate(hq, h, node_p, nb_p, w_p_flat):
    mesh = plsc.VectorSubcoreMesh(core_axis_name="c", subcore_axis_name="s")
    fn = functools.partial(
        pl.kernel,
        out_type=(
            jax.ShapeDtypeStruct((_BPAD, _D), jnp.float32),
            jax.ShapeDtypeStruct((_BPAD * _D,), jnp.float32),
        ),
        mesh=mesh,
        scratch_types=[
            pltpu.VMEM((_BPW,), jnp.int32),
            pltpu.VMEM((_BPW, _T), jnp.int32),
            pltpu.VMEM((_BPW * _T,), jnp.float32),
            pltpu.VMEM((_BPW // 2, _D), jnp.float32),
            pltpu.VMEM((_BPW // 2 * _D,), jnp.float32),
            pltpu.VMEM((2, _T, _D), jnp.float32),
            pltpu.SemaphoreType.DMA,
            pltpu.SemaphoreType.DMA,
            pltpu.SemaphoreType.DMA,
            pltpu.SemaphoreType.DMA,
            pltpu.SemaphoreType.DMA,
        ],
    )(_sc_agg_body)
    return fn(hq, h, node_p, nb_p, w_p_flat)


def _sc_perm_body(src_hbm, perm_hbm, out_hbm, idx_v, rows_v, sem):
    base = _wid() * _BPW
    pltpu.sync_copy(perm_hbm.at[pl.ds(base, _BPW)], idx_v)
    for cp in _chunked_row_gather(src_hbm, idx_v, 0, rows_v, _BPW, sem):
        cp.wait()
    pltpu.sync_copy(rows_v, out_hbm.at[pl.ds(base, _BPW)])


def _sc_perm_gather(h_new, perm_p):
    mesh = plsc.VectorSubcoreMesh(core_axis_name="c", subcore_axis_name="s")
    fn = functools.partial(
        pl.kernel,
        out_type=jax.ShapeDtypeStruct((_BPAD, _D), jnp.float32),
        mesh=mesh,
        scratch_types=[
            pltpu.VMEM((_BPW,), jnp.int32),
            pltpu.VMEM((_BPW, _D), jnp.float32),
            pltpu.SemaphoreType.DMA,
        ],
    )(_sc_perm_body)
    return fn(h_new, perm_p)


def kernel(h, nodeset, nb_nodes, nb_weights, Q_w, Q_b, W_w, W_b):
    b, t = nb_nodes.shape
    pad = _BPAD - b
    win = jnp.zeros((_N,), jnp.int32).at[nodeset].max(
        jnp.arange(b, dtype=jnp.int32))
    perm = win[nodeset]
    spread = jnp.arange(pad, dtype=jnp.int32)
    perm_p = jnp.concatenate([perm, spread])
    node_p = jnp.concatenate([nodeset, spread])
    nb_p = jnp.concatenate(
        [nb_nodes,
         (jnp.arange(pad * t, dtype=jnp.int32) % _N).reshape(pad, t)])
    w_p_flat = jnp.concatenate(
        [nb_weights.reshape(-1), jnp.ones((pad * t,), jnp.float32)])

    hq = _hq_precompute(h, Q_w.T, Q_b.reshape(1, _D))
    hn, agg_flat = _sc_aggregate(hq, h, node_p, nb_p, w_p_flat)
    agg = agg_flat.reshape(_BPAD, _D)
    h_new = _out_layer(hn, agg, w_p_flat.reshape(_BPAD, _T),
                       W_w[:, :_D].T, W_w[:, _D:].T, W_b.reshape(1, _D))
    out = _sc_perm_gather(h_new, perm_p)
    return out[:b]
